# baseline (speedup 1.0000x reference)
import functools

import jax
import jax.numpy as jnp
from jax import lax
from jax.experimental import pallas as pl
from jax.experimental.pallas import tpu as pltpu
from jax.experimental.pallas import tpu_sc as plsc

NUM_STOCKS = 1000000
CELL_SIZE = 64
BATCH = 16384

_info = plsc.get_sparse_core_info()
_NC, _NS = _info.num_cores, _info.num_subcores
_NW = _NC * _NS
_B_PER_W = BATCH // _NW


_mesh = plsc.VectorSubcoreMesh(core_axis_name="c", subcore_axis_name="s")


@functools.partial(
    pl.kernel,
    mesh=_mesh,
    compiler_params=pltpu.CompilerParams(use_tc_tiling_on_sc=False),
    out_type=(
        jax.ShapeDtypeStruct((BATCH, CELL_SIZE), jnp.float32),
        jax.ShapeDtypeStruct((BATCH, CELL_SIZE), jnp.float32),
    ),
    scratch_types=[
        pltpu.VMEM((_B_PER_W,), jnp.int32),
        pltpu.VMEM((_B_PER_W, CELL_SIZE), jnp.float32),
        pltpu.VMEM((_B_PER_W, CELL_SIZE), jnp.float32),
        pltpu.SemaphoreType.DMA,
        pltpu.SemaphoreType.DMA,
    ],
)
def _gather2(idx_hbm, t0_hbm, t1_hbm, out0_hbm, out1_hbm,
             idx_v, rows0_v, rows1_v, sem0, sem1):
    wid = lax.axis_index("s") * _NC + lax.axis_index("c")
    base = wid * _B_PER_W
    pltpu.sync_copy(idx_hbm.at[pl.ds(base, _B_PER_W)], idx_v)
    c0 = pltpu.async_copy(t0_hbm.at[idx_v], rows0_v, sem0)
    c1 = pltpu.async_copy(t1_hbm.at[idx_v], rows1_v, sem1)
    c0.wait()
    c1.wait()
    pltpu.sync_copy(rows0_v, out0_hbm.at[pl.ds(base, _B_PER_W)])
    pltpu.sync_copy(rows1_v, out1_hbm.at[pl.ds(base, _B_PER_W)])


def kernel(stock_id, table0, table1):
    idx = stock_id.reshape(BATCH).astype(jnp.int32)
    state0, state1 = _gather2(idx, table0, table1)
    return state0, state1


# speedup vs baseline: 1.5605x; 1.5605x over previous
import functools

import jax
import jax.numpy as jnp
from jax import lax
from jax.experimental import pallas as pl
from jax.experimental.pallas import tpu as pltpu
from jax.experimental.pallas import tpu_sc as plsc

NUM_STOCKS = 1000000
CELL_SIZE = 64
BATCH = 16384

_info = plsc.get_sparse_core_info()
_NC, _NS, _NL = _info.num_cores, _info.num_subcores, _info.num_lanes
_NW = _NC * _NS
_B_PER_W = BATCH // _NW
_CHUNK = 64
_N_CHUNKS = _B_PER_W // _CHUNK

_mesh = plsc.VectorSubcoreMesh(core_axis_name="c", subcore_axis_name="s")


@functools.partial(
    pl.kernel,
    mesh=_mesh,
    out_type=(
        jax.ShapeDtypeStruct((BATCH, CELL_SIZE), jnp.float32),
        jax.ShapeDtypeStruct((BATCH, CELL_SIZE), jnp.float32),
    ),
    scratch_types=[
        pltpu.VMEM((_B_PER_W,), jnp.int32),
        pltpu.VMEM((_CHUNK, CELL_SIZE), jnp.float32),
        pltpu.SemaphoreType.DMA,
        pltpu.SemaphoreType.DMA,
    ],
)
def _gather2(idx_hbm, t0_hbm, t1_hbm, out0_hbm, out1_hbm,
             idx_v, rows_v, sem, sem_out):
    wid = lax.axis_index("s") * _NC + lax.axis_index("c")
    base = wid * _B_PER_W
    pltpu.sync_copy(idx_hbm.at[pl.ds(base, _B_PER_W)], idx_v)

    for t_hbm, out_hbm in ((t0_hbm, out0_hbm), (t1_hbm, out1_hbm)):

        def chunk_body(ch, _, t_hbm=t_hbm, out_hbm=out_hbm):
            c0 = ch * _CHUNK

            for g in range(_CHUNK // _NL):
                v = idx_v[pl.ds(c0 + g * _NL, _NL)]
                for k in range(_NL):
                    i = v[k]
                    pltpu.async_copy(
                        t_hbm.at[pl.ds(i, 1), :],
                        rows_v.at[pl.ds(g * _NL + k, 1), :], sem)
            pltpu.make_async_copy(
                t_hbm.at[pl.ds(0, _CHUNK), :], rows_v, sem).wait()
            pltpu.async_copy(
                rows_v, out_hbm.at[pl.ds(base + c0, _CHUNK)], sem_out).wait()
            return _

        lax.fori_loop(0, _N_CHUNKS, chunk_body, None)


def kernel(stock_id, table0, table1):
    idx = stock_id.reshape(BATCH).astype(jnp.int32)
    return _gather2(idx, table0, table1)


# speedup vs baseline: 2.1955x; 1.4069x over previous
import functools

import jax
import jax.numpy as jnp
from jax import lax
from jax.experimental import pallas as pl
from jax.experimental.pallas import tpu as pltpu
from jax.experimental.pallas import tpu_sc as plsc

NUM_STOCKS = 1000000
CELL_SIZE = 64
BATCH = 16384

_info = plsc.get_sparse_core_info()
_NC, _NS, _NL = _info.num_cores, _info.num_subcores, _info.num_lanes
_NW = _NC * _NS
_B_PER_W = BATCH // _NW
_CHUNK = 64
_N_CHUNKS = _B_PER_W // _CHUNK
_NBUF = 4
_TAIL_START = (NUM_STOCKS // 128) * 128

_mesh = plsc.VectorSubcoreMesh(core_axis_name="c", subcore_axis_name="s")


@functools.partial(
    pl.kernel,
    mesh=_mesh,
    compiler_params=pltpu.CompilerParams(needs_layout_passes=False),
    out_type=(
        jax.ShapeDtypeStruct((BATCH, 128), jnp.float32),
        jax.ShapeDtypeStruct((BATCH, 128), jnp.float32),
    ),
    scratch_types=[
        pltpu.VMEM((_B_PER_W,), jnp.int32),
        [pltpu.VMEM((CELL_SIZE, 128), jnp.float32) for _ in range(_NBUF)],
        pltpu.VMEM((_CHUNK, 128), jnp.float32),
        [pltpu.SemaphoreType.DMA for _ in range(_NBUF)],
    ],
)
def _gather2t(idx_hbm, t0_hbm, t1_hbm, out0_hbm, out1_hbm,
              idx_v, bufs, rows_v, sems):
    wid = lax.axis_index("s") * _NC + lax.axis_index("c")
    base = wid * _B_PER_W
    pltpu.sync_copy(idx_hbm.at[pl.ds(base, _B_PER_W)], idx_v)

    r_vecs = [lax.iota(jnp.int32, _NL) + m * _NL
              for m in range(CELL_SIZE // _NL)]

    for t_hbm, out_hbm in ((t0_hbm, out0_hbm), (t1_hbm, out1_hbm)):

        def chunk_body(ch, _, t_hbm=t_hbm, out_hbm=out_hbm):
            c0 = ch * _CHUNK
            scalars = []
            for g in range(_CHUNK // _NL):
                v = idx_v[pl.ds(c0 + g * _NL, _NL)]
                for k in range(_NL):
                    scalars.append(v[k])

            def fire(j):
                i = scalars[j]
                off = pl.multiple_of(
                    lax.shift_left(lax.shift_right_logical(i, 7), 7), 128)
                pltpu.async_copy(
                    t_hbm.at[:, pl.ds(off, 128)], bufs[j % _NBUF],
                    sems[j % _NBUF])

            for j in range(_NBUF):
                fire(j)
            for j in range(_CHUNK):
                b = j % _NBUF
                pltpu.make_async_copy(
                    t_hbm.at[:, pl.ds(0, 128)], bufs[b], sems[b]).wait()
                c = scalars[j] & 127
                c_v = jnp.full((_NL,), c, jnp.int32)
                for m in range(CELL_SIZE // _NL):
                    vals = plsc.load_gather(bufs[b], [r_vecs[m], c_v])
                    rows_v[j, pl.ds(m * _NL, _NL)] = vals
                if j + _NBUF < _CHUNK:
                    fire(j + _NBUF)
            pltpu.sync_copy(rows_v, out_hbm.at[pl.ds(base + c0, _CHUNK), :])
            return _

        lax.fori_loop(0, _N_CHUNKS, chunk_body, None)


def _patch_tail(idx, out, table):
    n_tail = NUM_STOCKS + 1 - _TAIL_START
    tail = table[_TAIL_START:]
    sel = (idx[:, None] == (jnp.arange(n_tail, dtype=idx.dtype)
                            + _TAIL_START)[None, :])
    patch = jnp.einsum("be,ec->bc", sel.astype(jnp.float32), tail)
    return jnp.where((idx >= _TAIL_START)[:, None], patch, out)


def kernel(stock_id, table0, table1):
    idx = stock_id.reshape(BATCH).astype(jnp.int32)
    o0, o1 = _gather2t(idx, table0.T, table1.T)
    o0 = _patch_tail(idx, o0[:, :CELL_SIZE], table0)
    o1 = _patch_tail(idx, o1[:, :CELL_SIZE], table1)
    return o0, o1


# speedup vs baseline: 2.5406x; 1.1572x over previous
import functools

import jax
import jax.numpy as jnp
from jax import lax
from jax.experimental import pallas as pl
from jax.experimental.pallas import tpu as pltpu
from jax.experimental.pallas import tpu_sc as plsc

NUM_STOCKS = 1000000
CELL_SIZE = 64
BATCH = 16384

_info = plsc.get_sparse_core_info()
_NC, _NS, _NL = _info.num_cores, _info.num_subcores, _info.num_lanes
_NW = _NC * _NS
_B_PER_W = BATCH // _NW
_CHUNK = 64
_N_CHUNKS = _B_PER_W // _CHUNK
_NBUF = 8
_TAIL_START = (NUM_STOCKS // 128) * 128

_mesh = plsc.VectorSubcoreMesh(core_axis_name="c", subcore_axis_name="s")


@functools.partial(
    pl.kernel,
    mesh=_mesh,
    compiler_params=pltpu.CompilerParams(needs_layout_passes=False),
    out_type=(
        jax.ShapeDtypeStruct((BATCH, 128), jnp.float32),
        jax.ShapeDtypeStruct((BATCH, 128), jnp.float32),
    ),
    scratch_types=[
        pltpu.VMEM((_B_PER_W,), jnp.int32),
        [pltpu.VMEM((CELL_SIZE, 128), jnp.float32) for _ in range(_NBUF)],
        pltpu.VMEM((_CHUNK, 128), jnp.float32),
        [pltpu.SemaphoreType.DMA for _ in range(_NBUF)],
    ],
)
def _gather2t(idx_hbm, t0_hbm, t1_hbm, out0_hbm, out1_hbm,
              idx_v, bufs, rows_v, sems):
    wid = lax.axis_index("s") * _NC + lax.axis_index("c")
    base = wid * _B_PER_W
    pltpu.sync_copy(idx_hbm.at[pl.ds(base, _B_PER_W)], idx_v)

    r_vecs = [lax.iota(jnp.int32, _NL) + m * _NL
              for m in range(CELL_SIZE // _NL)]

    for t_hbm, out_hbm in ((t0_hbm, out0_hbm), (t1_hbm, out1_hbm)):

        def chunk_body(ch, _, t_hbm=t_hbm, out_hbm=out_hbm):
            c0 = ch * _CHUNK
            scalars = []
            for g in range(_CHUNK // _NL):
                v = idx_v[pl.ds(c0 + g * _NL, _NL)]
                for k in range(_NL):
                    scalars.append(v[k])

            def fire(j):
                i = scalars[j]
                off = pl.multiple_of(
                    lax.shift_left(lax.shift_right_logical(i, 7), 7), 128)
                pltpu.async_copy(
                    t_hbm.at[:, pl.ds(off, 128)], bufs[j % _NBUF],
                    sems[j % _NBUF])

            for j in range(_NBUF):
                fire(j)
            for j in range(_CHUNK):
                b = j % _NBUF
                pltpu.make_async_copy(
                    t_hbm.at[:, pl.ds(0, 128)], bufs[b], sems[b]).wait()
                c = scalars[j] & 127
                c_v = jnp.full((_NL,), c, jnp.int32)
                for m in range(CELL_SIZE // _NL):
                    vals = plsc.load_gather(bufs[b], [r_vecs[m], c_v])
                    rows_v[j, pl.ds(m * _NL, _NL)] = vals
                if j + _NBUF < _CHUNK:
                    fire(j + _NBUF)
            pltpu.sync_copy(rows_v, out_hbm.at[pl.ds(base + c0, _CHUNK), :])
            return _

        lax.fori_loop(0, _N_CHUNKS, chunk_body, None)


def _patch_tail(idx, out, table):
    n_tail = NUM_STOCKS + 1 - _TAIL_START
    tail = table[_TAIL_START:]
    sel = (idx[:, None] == (jnp.arange(n_tail, dtype=idx.dtype)
                            + _TAIL_START)[None, :])
    patch = jnp.einsum("be,ec->bc", sel.astype(jnp.float32), tail)
    return jnp.where((idx >= _TAIL_START)[:, None], patch, out)


def kernel(stock_id, table0, table1):
    idx = stock_id.reshape(BATCH).astype(jnp.int32)
    o0, o1 = _gather2t(idx, table0.T, table1.T)
    o0 = _patch_tail(idx, o0[:, :CELL_SIZE], table0)
    o1 = _patch_tail(idx, o1[:, :CELL_SIZE], table1)
    return o0, o1
